# baseline (speedup 1.0000x reference)
import functools

import jax
import jax.numpy as jnp
from jax import lax
from jax.experimental import pallas as pl
from jax.experimental.pallas import tpu as pltpu
from jax.experimental.pallas import tpu_sc as plsc

BATCH = 16384
DIM = 64
NLANE = 16
NVREG = DIM // NLANE

_info = plsc.get_sparse_core_info()
NC, NS = _info.num_cores, _info.num_subcores
NW = NC * NS
W = BATCH // NW
C = 128
NCH = W // C
UNROLL = 8


def _allsum(v):
    lanes = lax.iota(jnp.int32, NLANE)
    for s in (8, 4, 2, 1):
        v = v + v.at[lanes ^ s].get(mode="promise_in_bounds")
    return v


def _sc_body(h_hbm, t_hbm, r_hbm, E_hbm, RC_hbm, out_hbm,
             idx_h, idx_t, idx_r, buf_h, buf_t, buf_rc, out_c, sem):
    wid = lax.axis_index("s") * NC + lax.axis_index("c")
    base = wid * W

    for j in range(NCH):
        off = base + j * C
        pltpu.sync_copy(h_hbm.at[pl.ds(off, C)], idx_h)
        pltpu.sync_copy(t_hbm.at[pl.ds(off, C)], idx_t)
        pltpu.sync_copy(r_hbm.at[pl.ds(off, C)], idx_r)
        cp_h = pltpu.async_copy(E_hbm.at[idx_h], buf_h, sem)
        cp_t = pltpu.async_copy(E_hbm.at[idx_t], buf_t, sem)
        cp_r = pltpu.async_copy(RC_hbm.at[idx_r], buf_rc, sem)
        cp_h.wait()
        cp_t.wait()
        cp_r.wait()

        def body(g, carry):
            i0 = g * UNROLL
            mask = lax.iota(jnp.int32, NLANE) == 0
            for u in range(UNROLL):
                i = i0 + u
                d = [buf_h[i, pl.ds(k * NLANE, NLANE)]
                     - buf_t[i, pl.ds(k * NLANE, NLANE)]
                     for k in range(NVREG)]
                acc = d[0] * buf_rc[i, pl.ds(0, NLANE)]
                for k in range(1, NVREG):
                    acc = acc + d[k] * buf_rc[i, pl.ds(k * NLANE, NLANE)]
                s = _allsum(acc)
                m = jnp.abs(d[0] + s + buf_rc[i, pl.ds(DIM, NLANE)])
                for k in range(1, NVREG):
                    m = m + jnp.abs(d[k] + s
                                    + buf_rc[i, pl.ds(DIM + k * NLANE, NLANE)])
                tot = _allsum(m)
                idx = jnp.full((NLANE,), i, jnp.int32)
                plsc.store_scatter(out_c, [idx], tot, mask=mask)
            return carry

        lax.fori_loop(0, C // UNROLL, body, 0)
        pltpu.sync_copy(out_c, out_hbm.at[pl.ds(off, C)])


@jax.jit
def kernel(h, r, t, E, R, R_proj):
    RC = jnp.concatenate([R_proj, R], axis=1)
    mesh = plsc.VectorSubcoreMesh(core_axis_name="c", subcore_axis_name="s")
    f = pl.kernel(
        _sc_body,
        out_type=jax.ShapeDtypeStruct((BATCH,), jnp.float32),
        mesh=mesh,
        compiler_params=pltpu.CompilerParams(
            needs_layout_passes=False, use_tc_tiling_on_sc=False),
        scratch_types=[
            pltpu.VMEM((C,), jnp.int32),
            pltpu.VMEM((C,), jnp.int32),
            pltpu.VMEM((C,), jnp.int32),
            pltpu.VMEM((C, DIM), jnp.float32),
            pltpu.VMEM((C, DIM), jnp.float32),
            pltpu.VMEM((C, 2 * DIM), jnp.float32),
            pltpu.VMEM((C,), jnp.float32),
            pltpu.SemaphoreType.DMA,
        ],
    )
    return f(h, t, r, E, RC)


# speedup vs baseline: 2.1756x; 2.1756x over previous
import jax
import jax.numpy as jnp
from jax import lax
from jax.experimental import pallas as pl
from jax.experimental.pallas import tpu as pltpu
from jax.experimental.pallas import tpu_sc as plsc

BATCH = 16384
DIM = 64
NLANE = 16
NVREG = DIM // NLANE

_info = plsc.get_sparse_core_info()
NC, NS = _info.num_cores, _info.num_subcores
NW = NC * NS
W = BATCH // NW
C = 128
NCH = W // C
GRP = NLANE
NGRP = C // GRP
NSLOT = 3


def _allsum(v):
    lanes = lax.iota(jnp.int32, NLANE)
    for s in (8, 4, 2, 1):
        v = v + v.at[lanes ^ s].get(mode="promise_in_bounds")
    return v


def _sc_body(h_hbm, t_hbm, r_hbm, E3_hbm, RC_hbm, out_hbm,
             idx_h, idx_t, idx_r, buf_h, buf_t, buf_rc, out_c, drain_v,
             sem0, sem1, sem2, sem_r):
    wid = lax.axis_index("s") * NC + lax.axis_index("c")
    base = wid * W
    lanes = lax.iota(jnp.int32, NLANE)
    store_mask = lanes == 0
    sems = [sem0, sem1, sem2]

    def fire(g):
        for par in range(NSLOT):
            @pl.when(g % NSLOT == par)
            def _():
                ihb = idx_h[pl.ds(g * GRP, GRP)] >> 3
                itb = idx_t[pl.ds(g * GRP, GRP)] >> 3
                for u in range(GRP):
                    pltpu.async_copy(
                        E3_hbm.at[ihb[u]], buf_h.at[par, u], sems[par])
                    pltpu.async_copy(
                        E3_hbm.at[itb[u]], buf_t.at[par, u], sems[par])

    def drain(g):
        for par in range(NSLOT):
            @pl.when(g % NSLOT == par)
            def _():
                for _u in range(2 * GRP):
                    pltpu.make_async_copy(
                        E3_hbm.at[0], drain_v, sems[par]).wait()

    def compute(g):
        i0 = g * GRP
        slot = g % NSLOT
        oh = idx_h[pl.ds(i0, NLANE)] & 7
        ot = idx_t[pl.ds(i0, NLANE)] & 7
        for u in range(GRP):
            i = i0 + u
            o_h = oh[u]
            o_t = ot[u]
            d = [buf_h[slot, u, o_h, pl.ds(k * NLANE, NLANE)]
                 - buf_t[slot, u, o_t, pl.ds(k * NLANE, NLANE)]
                 for k in range(NVREG)]
            acc = d[0] * buf_rc[i, pl.ds(0, NLANE)]
            for k in range(1, NVREG):
                acc = acc + d[k] * buf_rc[i, pl.ds(k * NLANE, NLANE)]
            s = _allsum(acc)
            m = jnp.abs(d[0] + s + buf_rc[i, pl.ds(DIM, NLANE)])
            for k in range(1, NVREG):
                m = m + jnp.abs(
                    d[k] + s + buf_rc[i, pl.ds(DIM + k * NLANE, NLANE)])
            tot = _allsum(m)
            idx = jnp.full((NLANE,), i, jnp.int32)
            plsc.store_scatter(out_c, [idx], tot, mask=store_mask)

    def chunk_body(j, carry):
        off = base + j * C
        pltpu.sync_copy(h_hbm.at[pl.ds(off, C)], idx_h)
        pltpu.sync_copy(t_hbm.at[pl.ds(off, C)], idx_t)
        pltpu.sync_copy(r_hbm.at[pl.ds(off, C)], idx_r)
        cp_r = pltpu.async_copy(RC_hbm.at[idx_r], buf_rc, sem_r)
        fire(0)
        fire(1)
        cp_r.wait()

        def group_body(g, carry2):
            @pl.when(g < NGRP - (NSLOT - 1))
            def _():
                fire(g + (NSLOT - 1))
            drain(g)
            compute(g)
            return carry2

        lax.fori_loop(0, NGRP, group_body, 0)
        pltpu.sync_copy(out_c, out_hbm.at[pl.ds(off, C)])
        return carry

    lax.fori_loop(0, NCH, chunk_body, 0)


@jax.jit
def kernel(h, r, t, E, R, R_proj):
    E3 = E.reshape(E.shape[0] // 8, 8, DIM)
    RC = jnp.concatenate([R_proj, R], axis=1)
    mesh = plsc.VectorSubcoreMesh(core_axis_name="c", subcore_axis_name="s")
    f = pl.kernel(
        _sc_body,
        out_type=jax.ShapeDtypeStruct((BATCH,), jnp.float32),
        mesh=mesh,
        compiler_params=pltpu.CompilerParams(needs_layout_passes=False),
        scratch_types=[
            pltpu.VMEM((C,), jnp.int32),
            pltpu.VMEM((C,), jnp.int32),
            pltpu.VMEM((C,), jnp.int32),
            pltpu.VMEM((NSLOT, GRP, 8, DIM), jnp.float32),
            pltpu.VMEM((NSLOT, GRP, 8, DIM), jnp.float32),
            pltpu.VMEM((C, 2 * DIM), jnp.float32),
            pltpu.VMEM((C,), jnp.float32),
            pltpu.VMEM((8, DIM), jnp.float32),
            pltpu.SemaphoreType.DMA,
            pltpu.SemaphoreType.DMA,
            pltpu.SemaphoreType.DMA,
            pltpu.SemaphoreType.DMA,
        ],
    )
    return f(h, t, r, E3, RC)
